# baseline (speedup 1.0000x reference)
import jax
import jax.numpy as jnp
from jax import lax
from jax.experimental import pallas as pl
from jax.experimental.pallas import tpu as pltpu
from jax.experimental.pallas import tpu_sc as plsc

N = 10000
N_PAD = 10240
E = 320000
NSUB = 16
NCORE = 2
BLK = 128
PIECE = 40
NB32 = 80
NB16 = 160
E_PAD = NCORE * NSUB * NB32 * BLK
SLAB = N_PAD // NSUB


def _pipe_piece(h_flat, src_v, dst_v, rows_a, rows_b, acc_sh, sema, semb):
    pltpu.async_copy(h_flat.at[src_v.at[0]], rows_a, sema)

    def step(t, carry):
        j0 = 2 * t
        j1 = j0 + 1
        pltpu.async_copy(h_flat.at[src_v.at[j1]], rows_b, semb)
        pltpu.make_async_copy(h_flat.at[src_v.at[j0]], rows_a, sema).wait()
        pltpu.sync_copy(rows_a, acc_sh.at[dst_v.at[j0]], add=True)
        j2 = jnp.minimum(j0 + 2, PIECE - 2)
        pltpu.async_copy(h_flat.at[src_v.at[j2]], rows_a, sema)
        pltpu.make_async_copy(h_flat.at[src_v.at[j1]], rows_b, semb).wait()
        pltpu.sync_copy(rows_b, acc_sh.at[dst_v.at[j1]], add=True)
        return carry

    lax.fori_loop(0, PIECE // 2, step, 0)
    pltpu.make_async_copy(h_flat.at[src_v.at[0]], rows_a, sema).wait()


def _make_segsum_c1():
    npp = NB32 // PIECE

    def body(table, src_h, dst_h, zrow, out,
             src_v, dst_v, rows_a, rows_b, acc_sh, sema, semb):
        cid = lax.axis_index("c")
        sid = lax.axis_index("s")
        wid = cid * NSUB + sid
        pltpu.sync_copy(zrow, acc_sh.at[pl.ds(sid * SLAB, SLAB)])
        plsc.subcore_barrier()
        for p in range(npp):
            pltpu.sync_copy(src_h.at[wid * npp + p], src_v)
            pltpu.sync_copy(dst_h.at[wid * npp + p], dst_v)
            _pipe_piece(table, src_v, dst_v, rows_a, rows_b, acc_sh,
                        sema, semb)
        plsc.subcore_barrier()
        base = cid * N_PAD + sid * SLAB
        pltpu.sync_copy(acc_sh.at[pl.ds(sid * SLAB, SLAB)],
                        out.at[pl.ds(base, SLAB)])

    mesh = plsc.VectorSubcoreMesh(core_axis_name="c", subcore_axis_name="s")
    return pl.kernel(
        body,
        out_type=jax.ShapeDtypeStruct((NCORE * N_PAD, 128), jnp.float32),
        mesh=mesh,
        scratch_types=[
            pltpu.VMEM((PIECE, BLK), jnp.int32),
            pltpu.VMEM((PIECE, BLK), jnp.int32),
            pltpu.VMEM((BLK, 128), jnp.float32),
            pltpu.VMEM((BLK, 128), jnp.float32),
            pltpu.VMEM_SHARED((N_PAD, 128), jnp.float32),
            pltpu.SemaphoreType.DMA,
            pltpu.SemaphoreType.DMA,
        ],
        name="sc_segsum_c1")


def _make_count():
    npp = NB32 // PIECE

    def body(dst_h, zrow, ones_h, out, dst_v, ones_v, acc_sh):
        cid = lax.axis_index("c")
        sid = lax.axis_index("s")
        wid = cid * NSUB + sid
        pltpu.sync_copy(ones_h, ones_v)
        pltpu.sync_copy(zrow, acc_sh.at[pl.ds(sid * SLAB, SLAB)])
        plsc.subcore_barrier()
        for p in range(npp):
            pltpu.sync_copy(dst_h.at[wid * npp + p], dst_v)

            def blk(j, carry):
                pltpu.sync_copy(ones_v, acc_sh.at[dst_v.at[j]], add=True)
                return carry

            lax.fori_loop(0, PIECE, blk, 0)
        plsc.subcore_barrier()
        base = cid * N_PAD + sid * SLAB
        pltpu.sync_copy(acc_sh.at[pl.ds(sid * SLAB, SLAB)],
                        out.at[pl.ds(base, SLAB)])

    mesh = plsc.VectorSubcoreMesh(core_axis_name="c", subcore_axis_name="s")
    return pl.kernel(
        body,
        out_type=jax.ShapeDtypeStruct((NCORE * N_PAD, 128), jnp.float32),
        mesh=mesh,
        scratch_types=[
            pltpu.VMEM((PIECE, BLK), jnp.int32),
            pltpu.VMEM((BLK, 128), jnp.float32),
            pltpu.VMEM_SHARED((N_PAD, 128), jnp.float32),
        ],
        name="sc_count")


def _make_segsum_c4():
    def body(h_flat, src_h, dst_h, zrow, out,
             src_v, dst_v, rows_a, rows_b, acc_sh, sema, semb):
        cid = lax.axis_index("c")
        sid = lax.axis_index("s")
        npp = NB16 // PIECE
        for k in range(2):
            ci = cid + NCORE * k
            pltpu.sync_copy(zrow, acc_sh.at[pl.ds(sid * SLAB, SLAB)])
            plsc.subcore_barrier()
            for p in range(npp):
                pltpu.sync_copy(src_h.at[ci * NSUB * npp + sid * npp + p],
                                src_v)
                pltpu.sync_copy(dst_h.at[sid * npp + p], dst_v)
                _pipe_piece(h_flat, src_v, dst_v, rows_a, rows_b, acc_sh,
                            sema, semb)
            plsc.subcore_barrier()
            pltpu.sync_copy(acc_sh.at[pl.ds(sid * SLAB, SLAB)],
                            out.at[pl.ds(ci * N_PAD + sid * SLAB, SLAB)])

    mesh = plsc.VectorSubcoreMesh(core_axis_name="c", subcore_axis_name="s")
    return pl.kernel(
        body,
        out_type=jax.ShapeDtypeStruct((4 * N_PAD, 128), jnp.float32),
        mesh=mesh,
        scratch_types=[
            pltpu.VMEM((PIECE, BLK), jnp.int32),
            pltpu.VMEM((PIECE, BLK), jnp.int32),
            pltpu.VMEM((BLK, 128), jnp.float32),
            pltpu.VMEM((BLK, 128), jnp.float32),
            pltpu.VMEM_SHARED((N_PAD, 128), jnp.float32),
            pltpu.SemaphoreType.DMA,
            pltpu.SemaphoreType.DMA,
        ],
        name="sc_segsum_c4")


_BN = 512


def _tc_r(C_h, F_out):
    def body(h, wr, out):
        bf = jnp.bfloat16
        accr = jnp.zeros((_BN, F_out), jnp.float32)
        for c in range(C_h):
            accr = accr + jnp.dot(h[c].astype(bf),
                                  wr[c * 128:(c + 1) * 128, :],
                                  preferred_element_type=jnp.float32)
        out[...] = accr

    return pl.pallas_call(
        body, grid=(N_PAD // _BN,),
        in_specs=[
            pl.BlockSpec((C_h, _BN, 128), lambda i: (0, i, 0)),
            pl.BlockSpec((C_h * 128, F_out), lambda i: (0, 0)),
        ],
        out_specs=pl.BlockSpec((_BN, F_out), lambda i: (i, 0)),
        out_shape=jax.ShapeDtypeStruct((N_PAD, F_out), jnp.float32),
        name=f"tc_r_h{C_h}")


def _tc_layer(C_agg, with_y):
    def body(*refs):
        if with_y:
            agg, cnt, wl, sb, r, wl4, out, yout = refs
        else:
            agg, cnt, wl, sb, r, out = refs
        bf = jnp.bfloat16
        acc = jnp.zeros((_BN, 512), jnp.float32)
        for c in range(C_agg):
            acc = acc + jnp.dot(agg[c].astype(bf),
                                wl[c * 128:(c + 1) * 128, :],
                                preferred_element_type=jnp.float32)
        cw = cnt[0] + cnt[1]
        inv = 1.0 / jnp.maximum(cw, 1.0)
        res = acc * inv + r[...]
        res = jnp.maximum(res * sb[0:1, :] + sb[1:2, :], 0.0)
        rows = pl.program_id(0) * _BN + lax.broadcasted_iota(
            jnp.int32, (_BN, 1), 0)
        res = jnp.where(rows < N, res, 0.0)
        for c in range(4):
            out[c] = res[:, c * 128:(c + 1) * 128]
        if with_y:
            yout[...] = jnp.dot(res.astype(bf), wl4[...],
                                preferred_element_type=jnp.float32)

    grid = (N_PAD // _BN,)
    in_specs = [
        pl.BlockSpec((C_agg, _BN, 128), lambda i: (0, i, 0)),
        pl.BlockSpec((2, _BN, 1), lambda i: (0, i, 0)),
        pl.BlockSpec((C_agg * 128, 512), lambda i: (0, 0)),
        pl.BlockSpec((2, 512), lambda i: (0, 0)),
        pl.BlockSpec((_BN, 512), lambda i: (i, 0)),
    ]
    out_shape = [jax.ShapeDtypeStruct((4, N_PAD, 128), jnp.float32)]
    out_specs = [pl.BlockSpec((4, _BN, 128), lambda i: (0, i, 0))]
    if with_y:
        in_specs.append(pl.BlockSpec((512, 128), lambda i: (0, 0)))
        out_shape.append(jax.ShapeDtypeStruct((N_PAD, 128), jnp.float32))
        out_specs.append(pl.BlockSpec((_BN, 128), lambda i: (i, 0)))

    return pl.pallas_call(
        body, grid=grid, in_specs=in_specs, out_specs=out_specs,
        out_shape=out_shape,
        name=f"tc_layer_a{C_agg}" + ("_y" if with_y else ""))


def _tc_final():
    def body(agg4, cnt, r4, b4, out):
        cw = cnt[0] + cnt[1]
        inv = 1.0 / jnp.maximum(cw, 1.0)
        out[...] = (agg4[0] + agg4[1]) * inv + b4[0:1, :] + r4[...]

    return pl.pallas_call(
        body, grid=(N_PAD // _BN,),
        in_specs=[
            pl.BlockSpec((2, _BN, 128), lambda i: (0, i, 0)),
            pl.BlockSpec((2, _BN, 1), lambda i: (0, i, 0)),
            pl.BlockSpec((_BN, 128), lambda i: (i, 0)),
            pl.BlockSpec((1, 128), lambda i: (0, 0)),
        ],
        out_specs=pl.BlockSpec((_BN, 128), lambda i: (i, 0)),
        out_shape=jax.ShapeDtypeStruct((N_PAD, 128), jnp.float32),
        name="tc_final")


def kernel(x, edge_index, Wl1, bl1, Wr1, g1, b1, Wl2, bl2, Wr2, g2, b2,
           Wl3, bl3, Wr3, g3, b3, Wl4, bl4, Wr4):
    f32 = jnp.float32
    src = edge_index[0]
    dst = edge_index[1]

    pad_e = E_PAD - E
    pad_idx = N + (jnp.arange(pad_e, dtype=jnp.int32) % (N_PAD - N))
    srcp = jnp.concatenate([src, pad_idx])
    dstp = jnp.concatenate([dst, pad_idx])
    npieces = E_PAD // (PIECE * BLK)
    dst32 = dstp.reshape(npieces, PIECE, BLK)
    src32 = srcp.reshape(npieces, PIECE, BLK)
    off4 = (jnp.arange(4, dtype=jnp.int32) * N_PAD)[:, None]
    src4c = (srcp[None, :] + off4).reshape(4 * npieces, PIECE, BLK)

    xp = jnp.pad(x, ((0, N_PAD - N), (0, 0)))

    bf16 = jnp.bfloat16

    def fold(Wl, bl, Wr, g, b):
        s = g / jnp.sqrt(f32(1.0 + 1e-5))
        return Wl.T.astype(bf16), Wr.T.astype(bf16), jnp.stack([s, bl * s + b])

    wl1t, wr1t, sb1 = fold(Wl1, bl1, Wr1, g1, b1)
    wl2t, wr2t, sb2 = fold(Wl2, bl2, Wr2, g2, b2)
    wl3t, wr3t, sb3 = fold(Wl3, bl3, Wr3, g3, b3)
    wl1d = jnp.concatenate([wl1t, wl1t], axis=0)
    wl4t = jnp.pad(Wl4.T, ((0, 0), (0, 7))).astype(bf16)
    wr4t = jnp.pad(Wr4.T, ((0, 0), (0, 7))).astype(bf16)
    b4 = jnp.pad(bl4, (0, 7))[None, :]

    z128 = jnp.zeros((SLAB, 128), f32)
    ones128 = jnp.ones((BLK, 128), f32)

    seg1 = _make_segsum_c1()
    segcnt = _make_count()
    seg4 = _make_segsum_c4()
    tcr1 = _tc_r(1, 512)
    tcr4 = _tc_r(4, 512)
    tcr4f = _tc_r(4, 128)
    tc1 = _tc_layer(2, False)
    tc2 = _tc_layer(4, False)
    tc3 = _tc_layer(4, True)
    tcf = _tc_final()

    cnt = segcnt(dst32, z128, ones128).reshape(2, N_PAD, 128)[:, :, :1]
    r1 = tcr1(xp.reshape(1, N_PAD, 128), wr1t)
    agg1 = seg1(xp, src32, dst32, z128)
    h1 = tc1(agg1.reshape(2, N_PAD, 128), cnt, wl1d, sb1, r1)[0]

    r2 = tcr4(h1, wr2t)
    agg2 = seg4(h1.reshape(4 * N_PAD, 128), src4c, dst32, z128)
    h2 = tc2(agg2.reshape(4, N_PAD, 128), cnt, wl2t, sb2, r2)[0]

    r3 = tcr4(h2, wr3t)
    agg3 = seg4(h2.reshape(4 * N_PAD, 128), src4c, dst32, z128)
    h3, y4 = tc3(agg3.reshape(4, N_PAD, 128), cnt, wl3t, sb3, r3, wl4t)

    r4 = tcr4f(h3, wr4t)
    agg4 = seg1(y4, src32, dst32, z128)
    out = tcf(agg4.reshape(2, N_PAD, 128), cnt, r4, b4)

    return out[:N, :121]


# speedup vs baseline: 1.0396x; 1.0396x over previous
import jax
import jax.numpy as jnp
from jax import lax
from jax.experimental import pallas as pl
from jax.experimental.pallas import tpu as pltpu
from jax.experimental.pallas import tpu_sc as plsc

N = 10000
N_PAD = 10240
E = 320000
NSUB = 16
NCORE = 2
BLK = 128
PIECE = 40
NB32 = 80
NB16 = 160
E_PAD = NCORE * NSUB * NB32 * BLK
SLAB = N_PAD // NSUB


def _pipe_piece(h_flat, src_v, dst_v, rows_a, rows_b, acc_sh, sema, semb):
    pltpu.async_copy(h_flat.at[src_v.at[0]], rows_a, sema)

    def step(t, carry):
        j0 = 2 * t
        j1 = j0 + 1
        pltpu.async_copy(h_flat.at[src_v.at[j1]], rows_b, semb)
        pltpu.make_async_copy(h_flat.at[src_v.at[j0]], rows_a, sema).wait()
        pltpu.sync_copy(rows_a, acc_sh.at[dst_v.at[j0]], add=True)
        j2 = jnp.minimum(j0 + 2, PIECE - 2)
        pltpu.async_copy(h_flat.at[src_v.at[j2]], rows_a, sema)
        pltpu.make_async_copy(h_flat.at[src_v.at[j1]], rows_b, semb).wait()
        pltpu.sync_copy(rows_b, acc_sh.at[dst_v.at[j1]], add=True)
        return carry

    lax.fori_loop(0, PIECE // 2, step, 0)
    pltpu.make_async_copy(h_flat.at[src_v.at[0]], rows_a, sema).wait()


def _make_segsum_c1():
    npp = NB32 // PIECE

    def body(table, src_h, dst_h, zrow, out,
             src_v, dst_v, rows_a, rows_b, acc_sh, sema, semb):
        cid = lax.axis_index("c")
        sid = lax.axis_index("s")
        wid = cid * NSUB + sid
        pltpu.sync_copy(zrow, acc_sh.at[pl.ds(sid * SLAB, SLAB)])
        plsc.subcore_barrier()
        for p in range(npp):
            pltpu.sync_copy(src_h.at[wid * npp + p], src_v)
            pltpu.sync_copy(dst_h.at[wid * npp + p], dst_v)
            _pipe_piece(table, src_v, dst_v, rows_a, rows_b, acc_sh,
                        sema, semb)
        plsc.subcore_barrier()
        base = cid * N_PAD + sid * SLAB
        pltpu.sync_copy(acc_sh.at[pl.ds(sid * SLAB, SLAB)],
                        out.at[pl.ds(base, SLAB)])

    mesh = plsc.VectorSubcoreMesh(core_axis_name="c", subcore_axis_name="s")
    return pl.kernel(
        body,
        out_type=jax.ShapeDtypeStruct((NCORE * N_PAD, 128), jnp.float32),
        mesh=mesh,
        scratch_types=[
            pltpu.VMEM((PIECE, BLK), jnp.int32),
            pltpu.VMEM((PIECE, BLK), jnp.int32),
            pltpu.VMEM((BLK, 128), jnp.float32),
            pltpu.VMEM((BLK, 128), jnp.float32),
            pltpu.VMEM_SHARED((N_PAD, 128), jnp.float32),
            pltpu.SemaphoreType.DMA,
            pltpu.SemaphoreType.DMA,
        ],
        name="sc_segsum_c1")


def _make_count():
    npp = NB32 // PIECE

    def body(dst_h, zrow, ones_h, out, dst_v, ones_v, acc_sh):
        cid = lax.axis_index("c")
        sid = lax.axis_index("s")
        wid = cid * NSUB + sid
        pltpu.sync_copy(ones_h, ones_v)
        pltpu.sync_copy(zrow, acc_sh.at[pl.ds(sid * SLAB, SLAB)])
        plsc.subcore_barrier()
        for p in range(npp):
            pltpu.sync_copy(dst_h.at[wid * npp + p], dst_v)

            def blk(j, carry):
                pltpu.sync_copy(ones_v, acc_sh.at[dst_v.at[j]], add=True)
                return carry

            lax.fori_loop(0, PIECE, blk, 0)
        plsc.subcore_barrier()
        base = cid * N_PAD + sid * SLAB
        pltpu.sync_copy(acc_sh.at[pl.ds(sid * SLAB, SLAB)],
                        out.at[pl.ds(base, SLAB)])

    mesh = plsc.VectorSubcoreMesh(core_axis_name="c", subcore_axis_name="s")
    return pl.kernel(
        body,
        out_type=jax.ShapeDtypeStruct((NCORE * N_PAD, 16), jnp.float32),
        mesh=mesh,
        scratch_types=[
            pltpu.VMEM((PIECE, BLK), jnp.int32),
            pltpu.VMEM((BLK, 16), jnp.float32),
            pltpu.VMEM_SHARED((N_PAD, 16), jnp.float32),
        ],
        name="sc_count")


def _make_segsum_c4():
    def body(h_flat, src_h, dst_h, zrow, out,
             src_v, dst_v, rows_a, rows_b, acc_sh, sema, semb):
        cid = lax.axis_index("c")
        sid = lax.axis_index("s")
        npp = NB16 // PIECE
        for k in range(2):
            ci = cid + NCORE * k
            pltpu.sync_copy(zrow, acc_sh.at[pl.ds(sid * SLAB, SLAB)])
            plsc.subcore_barrier()
            for p in range(npp):
                pltpu.sync_copy(src_h.at[ci * NSUB * npp + sid * npp + p],
                                src_v)
                pltpu.sync_copy(dst_h.at[sid * npp + p], dst_v)
                _pipe_piece(h_flat, src_v, dst_v, rows_a, rows_b, acc_sh,
                            sema, semb)
            plsc.subcore_barrier()
            pltpu.sync_copy(acc_sh.at[pl.ds(sid * SLAB, SLAB)],
                            out.at[pl.ds(ci * N_PAD + sid * SLAB, SLAB)])

    mesh = plsc.VectorSubcoreMesh(core_axis_name="c", subcore_axis_name="s")
    return pl.kernel(
        body,
        out_type=jax.ShapeDtypeStruct((4 * N_PAD, 128), jnp.float32),
        mesh=mesh,
        scratch_types=[
            pltpu.VMEM((PIECE, BLK), jnp.int32),
            pltpu.VMEM((PIECE, BLK), jnp.int32),
            pltpu.VMEM((BLK, 128), jnp.float32),
            pltpu.VMEM((BLK, 128), jnp.float32),
            pltpu.VMEM_SHARED((N_PAD, 128), jnp.float32),
            pltpu.SemaphoreType.DMA,
            pltpu.SemaphoreType.DMA,
        ],
        name="sc_segsum_c4")


_BN = 512


def _tc_r(C_h, F_out):
    def body(h, wr, out):
        bf = jnp.bfloat16
        accr = jnp.zeros((_BN, F_out), jnp.float32)
        for c in range(C_h):
            accr = accr + jnp.dot(h[c].astype(bf),
                                  wr[c * 128:(c + 1) * 128, :],
                                  preferred_element_type=jnp.float32)
        out[...] = accr

    return pl.pallas_call(
        body, grid=(N_PAD // _BN,),
        in_specs=[
            pl.BlockSpec((C_h, _BN, 128), lambda i: (0, i, 0)),
            pl.BlockSpec((C_h * 128, F_out), lambda i: (0, 0)),
        ],
        out_specs=pl.BlockSpec((_BN, F_out), lambda i: (i, 0)),
        out_shape=jax.ShapeDtypeStruct((N_PAD, F_out), jnp.float32),
        name=f"tc_r_h{C_h}")


def _tc_layer(C_agg, with_y):
    def body(*refs):
        if with_y:
            agg, cnt, wl, sb, r, wl4, out, yout = refs
        else:
            agg, cnt, wl, sb, r, out = refs
        bf = jnp.bfloat16
        acc = jnp.zeros((_BN, 512), jnp.float32)
        for c in range(C_agg):
            acc = acc + jnp.dot(agg[c].astype(bf),
                                wl[c * 128:(c + 1) * 128, :],
                                preferred_element_type=jnp.float32)
        cw = cnt[0] + cnt[1]
        inv = 1.0 / jnp.maximum(cw, 1.0)
        res = acc * inv + r[...]
        res = jnp.maximum(res * sb[0:1, :] + sb[1:2, :], 0.0)
        rows = pl.program_id(0) * _BN + lax.broadcasted_iota(
            jnp.int32, (_BN, 1), 0)
        res = jnp.where(rows < N, res, 0.0)
        for c in range(4):
            out[c] = res[:, c * 128:(c + 1) * 128]
        if with_y:
            yout[...] = jnp.dot(res.astype(bf), wl4[...],
                                preferred_element_type=jnp.float32)

    grid = (N_PAD // _BN,)
    in_specs = [
        pl.BlockSpec((C_agg, _BN, 128), lambda i: (0, i, 0)),
        pl.BlockSpec((2, _BN, 1), lambda i: (0, i, 0)),
        pl.BlockSpec((C_agg * 128, 512), lambda i: (0, 0)),
        pl.BlockSpec((2, 512), lambda i: (0, 0)),
        pl.BlockSpec((_BN, 512), lambda i: (i, 0)),
    ]
    out_shape = [jax.ShapeDtypeStruct((4, N_PAD, 128), jnp.float32)]
    out_specs = [pl.BlockSpec((4, _BN, 128), lambda i: (0, i, 0))]
    if with_y:
        in_specs.append(pl.BlockSpec((512, 128), lambda i: (0, 0)))
        out_shape.append(jax.ShapeDtypeStruct((N_PAD, 128), jnp.float32))
        out_specs.append(pl.BlockSpec((_BN, 128), lambda i: (i, 0)))

    return pl.pallas_call(
        body, grid=grid, in_specs=in_specs, out_specs=out_specs,
        out_shape=out_shape,
        name=f"tc_layer_a{C_agg}" + ("_y" if with_y else ""))


def _tc_final():
    def body(agg4, cnt, r4, b4, out):
        cw = cnt[0] + cnt[1]
        inv = 1.0 / jnp.maximum(cw, 1.0)
        out[...] = (agg4[0] + agg4[1]) * inv + b4[0:1, :] + r4[...]

    return pl.pallas_call(
        body, grid=(N_PAD // _BN,),
        in_specs=[
            pl.BlockSpec((2, _BN, 128), lambda i: (0, i, 0)),
            pl.BlockSpec((2, _BN, 1), lambda i: (0, i, 0)),
            pl.BlockSpec((_BN, 128), lambda i: (i, 0)),
            pl.BlockSpec((1, 128), lambda i: (0, 0)),
        ],
        out_specs=pl.BlockSpec((_BN, 128), lambda i: (i, 0)),
        out_shape=jax.ShapeDtypeStruct((N_PAD, 128), jnp.float32),
        name="tc_final")


def kernel(x, edge_index, Wl1, bl1, Wr1, g1, b1, Wl2, bl2, Wr2, g2, b2,
           Wl3, bl3, Wr3, g3, b3, Wl4, bl4, Wr4):
    f32 = jnp.float32
    src = edge_index[0]
    dst = edge_index[1]

    pad_e = E_PAD - E
    pad_idx = N + (jnp.arange(pad_e, dtype=jnp.int32) % (N_PAD - N))
    srcp = jnp.concatenate([src, pad_idx])
    dstp = jnp.concatenate([dst, pad_idx])
    npieces = E_PAD // (PIECE * BLK)
    dst32 = dstp.reshape(npieces, PIECE, BLK)
    src32 = srcp.reshape(npieces, PIECE, BLK)
    off4 = (jnp.arange(4, dtype=jnp.int32) * N_PAD)[:, None]
    src4c = (srcp[None, :] + off4).reshape(4 * npieces, PIECE, BLK)

    xp = jnp.pad(x, ((0, N_PAD - N), (0, 0)))

    bf16 = jnp.bfloat16

    def fold(Wl, bl, Wr, g, b):
        s = g / jnp.sqrt(f32(1.0 + 1e-5))
        return Wl.T.astype(bf16), Wr.T.astype(bf16), jnp.stack([s, bl * s + b])

    wl1t, wr1t, sb1 = fold(Wl1, bl1, Wr1, g1, b1)
    wl2t, wr2t, sb2 = fold(Wl2, bl2, Wr2, g2, b2)
    wl3t, wr3t, sb3 = fold(Wl3, bl3, Wr3, g3, b3)
    wl1d = jnp.concatenate([wl1t, wl1t], axis=0)
    wl4t = jnp.pad(Wl4.T, ((0, 0), (0, 7))).astype(bf16)
    wr4t = jnp.pad(Wr4.T, ((0, 0), (0, 7))).astype(bf16)
    b4 = jnp.pad(bl4, (0, 7))[None, :]

    z128 = jnp.zeros((SLAB, 128), f32)
    z16 = jnp.zeros((SLAB, 16), f32)
    ones16 = jnp.ones((BLK, 16), f32)

    seg1 = _make_segsum_c1()
    segcnt = _make_count()
    seg4 = _make_segsum_c4()
    tcr1 = _tc_r(1, 512)
    tcr4 = _tc_r(4, 512)
    tcr4f = _tc_r(4, 128)
    tc1 = _tc_layer(2, False)
    tc2 = _tc_layer(4, False)
    tc3 = _tc_layer(4, True)
    tcf = _tc_final()

    cnt = segcnt(dst32, z16, ones16).reshape(2, N_PAD, 16)[:, :, :1]
    r1 = tcr1(xp.reshape(1, N_PAD, 128), wr1t)
    agg1 = seg1(xp, src32, dst32, z128)
    h1 = tc1(agg1.reshape(2, N_PAD, 128), cnt, wl1d, sb1, r1)[0]

    r2 = tcr4(h1, wr2t)
    agg2 = seg4(h1.reshape(4 * N_PAD, 128), src4c, dst32, z128)
    h2 = tc2(agg2.reshape(4, N_PAD, 128), cnt, wl2t, sb2, r2)[0]

    r3 = tcr4(h2, wr3t)
    agg3 = seg4(h2.reshape(4 * N_PAD, 128), src4c, dst32, z128)
    h3, y4 = tc3(agg3.reshape(4, N_PAD, 128), cnt, wl3t, sb3, r3, wl4t)

    r4 = tcr4f(h3, wr4t)
    agg4 = seg1(y4, src32, dst32, z128)
    out = tcf(agg4.reshape(2, N_PAD, 128), cnt, r4, b4)

    return out[:N, :121]
